# Initial kernel scaffold; baseline (speedup 1.0000x reference)
import jax
import jax.numpy as jnp
from jax.experimental import pallas as pl


def kernel(x, indices):
    raise NotImplementedError("write your pallas kernel here")


# speedup vs baseline: 1.2547x; 1.2547x over previous
import functools

import jax
import jax.numpy as jnp
from jax import lax
from jax.experimental import pallas as pl
from jax.experimental.pallas import tpu as pltpu
from jax.experimental.pallas import tpu_sc as plsc

_LANES = 16
_CHUNK = 8


def _connector_sc(x_flat, indices, *, n_rows, n_idx, d):
    num_workers = 32
    rows_per_batch = n_rows // num_workers
    n_chunks = n_idx // _CHUNK
    mesh = plsc.VectorSubcoreMesh(core_axis_name="c", subcore_axis_name="s")

    @functools.partial(
        pl.kernel,
        mesh=mesh,
        out_type=jax.ShapeDtypeStruct((num_workers * n_idx, d), jnp.float32),
        scratch_types=[
            pltpu.VMEM((n_idx,), jnp.int32),
            pltpu.VMEM((2, _CHUNK, d), jnp.float32),
            pltpu.SemaphoreType.DMA,
            pltpu.SemaphoreType.DMA,
        ],
    )
    def k(x_hbm, idx_hbm, out_hbm, idx_v, rows_v, gsem, ssem):
        wid = lax.axis_index("s") * 2 + lax.axis_index("c")
        pltpu.sync_copy(idx_hbm, idx_v)
        row_base = wid * rows_per_batch
        for i in range(n_idx // _LANES):
            sl = pl.ds(i * _LANES, _LANES)
            idx_v[sl] = idx_v[sl] + row_base

        out_base = wid * n_idx

        def gather(c, buf):
            return pltpu.async_copy(
                x_hbm.at[idx_v.at[pl.ds(c * _CHUNK, _CHUNK)]],
                rows_v.at[buf],
                gsem,
            )

        def scatter(c, buf):
            return pltpu.async_copy(
                rows_v.at[buf],
                out_hbm.at[pl.ds(out_base + c * _CHUNK, _CHUNK)],
                ssem,
            )

        g = [None] * n_chunks
        s = [None] * n_chunks
        g[0] = gather(0, 0)
        for c in range(n_chunks):
            buf = c & 1
            if c + 1 < n_chunks:
                if c >= 1:
                    s[c - 1].wait()
                g[c + 1] = gather(c + 1, buf ^ 1)
            g[c].wait()
            s[c] = scatter(c, buf)
        s[n_chunks - 2].wait()
        s[n_chunks - 1].wait()

    return k(x_flat, indices)


def kernel(x, indices):
    b, c, d = x.shape
    (n_idx,) = indices.shape
    x_flat = x.reshape(b * c, d)
    out_flat = _connector_sc(x_flat, indices, n_rows=b * c, n_idx=n_idx, d=d)
    return out_flat.reshape(b, n_idx, d)


# speedup vs baseline: 1.2744x; 1.0157x over previous
import functools

import jax
import jax.numpy as jnp
from jax import lax
from jax.experimental import pallas as pl
from jax.experimental.pallas import tpu as pltpu
from jax.experimental.pallas import tpu_sc as plsc

_LANES = 16
_CHUNK = 8
_NBUF = 2


def _connector_sc(x_flat, indices, *, n_rows, n_idx, d):
    num_workers = 32
    rows_per_batch = n_rows // num_workers
    n_chunks = n_idx // _CHUNK
    assert n_chunks % _NBUF == 0 and n_chunks >= 2 * _NBUF
    mesh = plsc.VectorSubcoreMesh(core_axis_name="c", subcore_axis_name="s")

    @functools.partial(
        pl.kernel,
        mesh=mesh,
        out_type=jax.ShapeDtypeStruct((num_workers * n_idx, d), jnp.float32),
        scratch_types=[
            pltpu.VMEM((n_idx,), jnp.int32),
            pltpu.VMEM((_NBUF, _CHUNK, d), jnp.float32),
            pltpu.SemaphoreType.DMA,
            pltpu.SemaphoreType.DMA,
        ],
    )
    def k(x_hbm, idx_hbm, out_hbm, idx_v, rows_v, gsem, ssem):
        wid = lax.axis_index("s") * 2 + lax.axis_index("c")
        pltpu.sync_copy(idx_hbm, idx_v)
        row_base = wid * rows_per_batch
        for i in range(n_idx // _LANES):
            sl = pl.ds(i * _LANES, _LANES)
            idx_v[sl] = idx_v[sl] + row_base

        out_base = wid * n_idx

        def gather(c, buf):
            return pltpu.async_copy(
                x_hbm.at[idx_v.at[pl.ds(c * _CHUNK, _CHUNK)]],
                rows_v.at[buf],
                gsem,
            )

        def scatter(c, buf):
            return pltpu.async_copy(
                rows_v.at[buf],
                out_hbm.at[pl.ds(out_base + c * _CHUNK, _CHUNK)],
                ssem,
            )

        def wait_gather(buf):
            pltpu.make_async_copy(x_hbm.at[pl.ds(0, _CHUNK)], rows_v.at[buf], gsem).wait()

        def wait_scatter(buf):
            pltpu.make_async_copy(
                rows_v.at[buf], out_hbm.at[pl.ds(out_base, _CHUNK)], ssem
            ).wait()

        for b in range(_NBUF):
            gather(b, b)

        @pl.loop(0, n_chunks - _NBUF, step=_NBUF)
        def _(c0):
            for b in range(_NBUF):
                c = c0 + b
                wait_gather(b)
                scatter(c, b)
                wait_scatter(b)
                gather(c + _NBUF, b)

        for b in range(_NBUF):
            c = n_chunks - _NBUF + b
            wait_gather(b)
            scatter(c, b)
        for b in range(_NBUF):
            wait_scatter(b)

    return k(x_flat, indices)


def kernel(x, indices):
    b, c, d = x.shape
    (n_idx,) = indices.shape
    x_flat = x.reshape(b * c, d)
    out_flat = _connector_sc(x_flat, indices, n_rows=b * c, n_idx=n_idx, d=d)
    return out_flat.reshape(b, n_idx, d)
